# baseline (speedup 1.0000x reference)
import functools

import jax
import jax.numpy as jnp
from jax.experimental import pallas as pl

_P1, _P2 = 778, 3000
_P1P, _P2P = 784, 3072
_PAD = 1e17


def _nn_body(x_ref, y_ref, n_ref, y2x_ref, x2y_ref, yidx_ref):
    xb = x_ref[0]
    yt = y_ref[0]
    nb = n_ref[0]

    xx = (
        xb[:, 0:1] * xb[:, 0:1]
        + xb[:, 1:2] * xb[:, 1:2]
        + xb[:, 2:3] * xb[:, 2:3]
    )
    yy = yt[0:1] * yt[0:1] + yt[1:2] * yt[1:2] + yt[2:3] * yt[2:3]
    xy2 = jax.lax.dot_general(
        xb * (-2.0),
        yt,
        (((1,), (0,)), ((), ())),
        preferred_element_type=jnp.float32,
        precision=jax.lax.Precision.HIGHEST,
    )
    d = (xx + xy2) + yy

    row_min = jnp.min(d, axis=1, keepdims=True)
    x2y_ref[0] = jnp.sqrt(jnp.maximum(row_min, 0.0))

    col_min = jnp.min(d, axis=0, keepdims=True)
    iota = jax.lax.broadcasted_iota(jnp.int32, d.shape, 0)
    yidx = jnp.min(jnp.where(d == col_min, iota, _P1P), axis=0, keepdims=True)
    yidx_ref[0] = yidx

    onehot = jnp.where(iota == yidx, 1.0, 0.0)
    a = (
        nb[:, 0:1] * xb[:, 0:1]
        + nb[:, 1:2] * xb[:, 1:2]
        + nb[:, 2:3] * xb[:, 2:3]
    )
    table = jnp.concatenate([nb, a], axis=1)
    gath = jax.lax.dot_general(
        table,
        onehot,
        (((0,), (0,)), ((), ())),
        preferred_element_type=jnp.float32,
        precision=jax.lax.Precision.HIGHEST,
    )
    dotv = (
        gath[0:1] * yt[0:1]
        + gath[1:2] * yt[1:2]
        + gath[2:3] * yt[2:3]
        - gath[3:4]
    )
    y2x_ref[0] = jnp.sqrt(jnp.maximum(col_min, 0.0)) * jnp.sign(dotv)


@functools.partial(jax.jit, static_argnames=())
def kernel(x, y, x_normals):
    B = x.shape[0]
    xp = jnp.pad(x, ((0, 0), (0, _P1P - _P1), (0, 0)), constant_values=_PAD)
    npad = jnp.pad(x_normals, ((0, 0), (0, _P1P - _P1), (0, 0)))
    yt = jnp.pad(
        jnp.transpose(y, (0, 2, 1)), ((0, 0), (0, 0), (0, _P2P - _P2)),
        constant_values=_PAD,
    )

    y2x_s, x2y_s, yidx = pl.pallas_call(
        _nn_body,
        grid=(B,),
        in_specs=[
            pl.BlockSpec((1, _P1P, 3), lambda b: (b, 0, 0)),
            pl.BlockSpec((1, 3, _P2P), lambda b: (b, 0, 0)),
            pl.BlockSpec((1, _P1P, 3), lambda b: (b, 0, 0)),
        ],
        out_specs=[
            pl.BlockSpec((1, 1, _P2P), lambda b: (b, 0, 0)),
            pl.BlockSpec((1, _P1P, 1), lambda b: (b, 0, 0)),
            pl.BlockSpec((1, 1, _P2P), lambda b: (b, 0, 0)),
        ],
        out_shape=[
            jax.ShapeDtypeStruct((B, 1, _P2P), jnp.float32),
            jax.ShapeDtypeStruct((B, _P1P, 1), jnp.float32),
            jax.ShapeDtypeStruct((B, 1, _P2P), jnp.int32),
        ],
    )(xp, yt, npad)

    return (
        y2x_s[:, 0, :_P2],
        x2y_s[:, :_P1, 0],
        yidx[:, 0, :_P2],
    )


# speedup vs baseline: 1.7999x; 1.7999x over previous
import functools

import jax
import jax.numpy as jnp
from jax.experimental import pallas as pl

_P1, _P2 = 778, 3000
_P1P, _P2P = 784, 3072
_PAD = 1e17


def _split3_bf16(t):
    hi = t.astype(jnp.bfloat16)
    r1 = t - hi.astype(jnp.float32)
    mid = r1.astype(jnp.bfloat16)
    lo = (r1 - mid.astype(jnp.float32)).astype(jnp.bfloat16)
    return hi, mid, lo


def _nn_body(x_ref, y_ref, n_ref, y2x_ref, x2y_ref, yidx_ref):
    xb = x_ref[0]
    yt = y_ref[0]
    nb = n_ref[0]

    d = None
    for c in range(3):
        diff = yt[c : c + 1, :] - xb[:, c : c + 1]
        sq = diff * diff
        d = sq if d is None else d + sq

    row_min = jnp.min(d, axis=1, keepdims=True)
    x2y_ref[0] = jnp.sqrt(row_min)

    col_min = jnp.min(d, axis=0, keepdims=True)
    onehot = jnp.where(d == col_min, 1.0, 0.0).astype(jnp.bfloat16)

    a = (
        nb[:, 0:1] * xb[:, 0:1]
        + nb[:, 1:2] * xb[:, 1:2]
        + nb[:, 2:3] * xb[:, 2:3]
    )
    ii = jax.lax.broadcasted_iota(jnp.int32, (_P1P, 1), 0).astype(jnp.float32)
    ones = jnp.ones((_P1P, 1), jnp.float32)
    table = jnp.concatenate([nb, a, ii, ones, ii * ii], axis=1)
    dims = (((0,), (0,)), ((), ()))
    gath = None
    for limb in _split3_bf16(table):
        part = jax.lax.dot_general(
            limb, onehot, dims, preferred_element_type=jnp.float32
        )
        gath = part if gath is None else gath + part

    s = gath[4:5]
    cnt = gath[5:6]
    q = gath[6:7]
    tie_lo = 0.5 * (s - jnp.sqrt(jnp.maximum(2.0 * q - s * s, 0.0)))
    yidx = jnp.where(cnt == 1.0, s, tie_lo)
    yidx_ref[0] = yidx.astype(jnp.int32)

    dotv = (
        gath[0:1] * yt[0:1]
        + gath[1:2] * yt[1:2]
        + gath[2:3] * yt[2:3]
        - gath[3:4]
    )
    y2x_ref[0] = jnp.sqrt(col_min) * jnp.sign(dotv)


@functools.partial(jax.jit, static_argnames=())
def kernel(x, y, x_normals):
    B = x.shape[0]
    xp = jnp.pad(x, ((0, 0), (0, _P1P - _P1), (0, 0)), constant_values=_PAD)
    npad = jnp.pad(x_normals, ((0, 0), (0, _P1P - _P1), (0, 0)))
    yt = jnp.pad(
        jnp.transpose(y, (0, 2, 1)), ((0, 0), (0, 0), (0, _P2P - _P2)),
        constant_values=_PAD,
    )

    y2x_s, x2y_s, yidx = pl.pallas_call(
        _nn_body,
        grid=(B,),
        in_specs=[
            pl.BlockSpec((1, _P1P, 3), lambda b: (b, 0, 0)),
            pl.BlockSpec((1, 3, _P2P), lambda b: (b, 0, 0)),
            pl.BlockSpec((1, _P1P, 3), lambda b: (b, 0, 0)),
        ],
        out_specs=[
            pl.BlockSpec((1, 1, _P2P), lambda b: (b, 0, 0)),
            pl.BlockSpec((1, _P1P, 1), lambda b: (b, 0, 0)),
            pl.BlockSpec((1, 1, _P2P), lambda b: (b, 0, 0)),
        ],
        out_shape=[
            jax.ShapeDtypeStruct((B, 1, _P2P), jnp.float32),
            jax.ShapeDtypeStruct((B, _P1P, 1), jnp.float32),
            jax.ShapeDtypeStruct((B, 1, _P2P), jnp.int32),
        ],
    )(xp, yt, npad)

    return (
        y2x_s[:, 0, :_P2],
        x2y_s[:, :_P1, 0],
        yidx[:, 0, :_P2],
    )


# speedup vs baseline: 1.8982x; 1.0546x over previous
import functools

import jax
import jax.numpy as jnp
from jax.experimental import pallas as pl

_P1, _P2 = 778, 3000
_P1P, _P2P = 784, 3072
_PAD = 1e17
_BPP = 2


def _split3_bf16(t):
    hi = t.astype(jnp.bfloat16)
    r1 = t - hi.astype(jnp.float32)
    mid = r1.astype(jnp.bfloat16)
    lo = (r1 - mid.astype(jnp.float32)).astype(jnp.bfloat16)
    return hi, mid, lo


def _nn_body(x_ref, y_ref, n_ref, y2x_ref, x2y_ref, yidx_ref):
  for s in range(_BPP):
      xb = x_ref[s]
      yt = y_ref[s]
      nb = n_ref[s]

      d = None
      for c in range(3):
          diff = yt[c : c + 1, :] - xb[:, c : c + 1]
          sq = diff * diff
          d = sq if d is None else d + sq

      row_min = jnp.min(d, axis=1, keepdims=True)
      x2y_ref[s] = jnp.sqrt(row_min)

      col_min = jnp.min(d, axis=0, keepdims=True)
      onehot = jnp.where(d == col_min, 1.0, 0.0).astype(jnp.bfloat16)

      a = (
          nb[:, 0:1] * xb[:, 0:1]
          + nb[:, 1:2] * xb[:, 1:2]
          + nb[:, 2:3] * xb[:, 2:3]
      )
      ii = jax.lax.broadcasted_iota(jnp.int32, (_P1P, 1), 0).astype(jnp.float32)
      ones = jnp.ones((_P1P, 1), jnp.float32)
      table = jnp.concatenate([nb, a, ii, ones, ii * ii], axis=1)
      dims = (((0,), (0,)), ((), ()))
      gath = None
      for limb in _split3_bf16(table):
          part = jax.lax.dot_general(
              limb, onehot, dims, preferred_element_type=jnp.float32
          )
          gath = part if gath is None else gath + part

      ssum = gath[4:5]
      cnt = gath[5:6]
      q = gath[6:7]
      tie_lo = 0.5 * (ssum - jnp.sqrt(jnp.maximum(2.0 * q - ssum * ssum, 0.0)))
      yidx = jnp.where(cnt == 1.0, ssum, tie_lo)
      yidx_ref[s] = yidx.astype(jnp.int32)

      dotv = (
          gath[0:1] * yt[0:1]
          + gath[1:2] * yt[1:2]
          + gath[2:3] * yt[2:3]
          - gath[3:4]
      )
      y2x_ref[s] = jnp.sqrt(col_min) * jnp.sign(dotv)


@functools.partial(jax.jit, static_argnames=())
def kernel(x, y, x_normals):
    B = x.shape[0]
    xp = jnp.pad(x, ((0, 0), (0, _P1P - _P1), (0, 0)), constant_values=_PAD)
    npad = jnp.pad(x_normals, ((0, 0), (0, _P1P - _P1), (0, 0)))
    yt = jnp.pad(
        jnp.transpose(y, (0, 2, 1)), ((0, 0), (0, 0), (0, _P2P - _P2)),
        constant_values=_PAD,
    )

    y2x_s, x2y_s, yidx = pl.pallas_call(
        _nn_body,
        grid=(B // _BPP,),
        in_specs=[
            pl.BlockSpec((_BPP, _P1P, 3), lambda b: (b, 0, 0)),
            pl.BlockSpec((_BPP, 3, _P2P), lambda b: (b, 0, 0)),
            pl.BlockSpec((_BPP, _P1P, 3), lambda b: (b, 0, 0)),
        ],
        out_specs=[
            pl.BlockSpec((_BPP, 1, _P2P), lambda b: (b, 0, 0)),
            pl.BlockSpec((_BPP, _P1P, 1), lambda b: (b, 0, 0)),
            pl.BlockSpec((_BPP, 1, _P2P), lambda b: (b, 0, 0)),
        ],
        out_shape=[
            jax.ShapeDtypeStruct((B, 1, _P2P), jnp.float32),
            jax.ShapeDtypeStruct((B, _P1P, 1), jnp.float32),
            jax.ShapeDtypeStruct((B, 1, _P2P), jnp.int32),
        ],
    )(xp, yt, npad)

    return (
        y2x_s[:, 0, :_P2],
        x2y_s[:, :_P1, 0],
        yidx[:, 0, :_P2],
    )


# speedup vs baseline: 2.0327x; 1.0709x over previous
import functools

import jax
import jax.numpy as jnp
from jax.experimental import pallas as pl

_P1, _P2 = 778, 3000
_BPP = 2


def _split3_bf16(t):
    hi = t.astype(jnp.bfloat16)
    r1 = t - hi.astype(jnp.float32)
    mid = r1.astype(jnp.bfloat16)
    lo = (r1 - mid.astype(jnp.float32)).astype(jnp.bfloat16)
    return hi, mid, lo


def _nn_body(x_ref, y_ref, n_ref, y2x_ref, x2y_ref, yidx_ref):
  for s in range(_BPP):
    xb = x_ref[s]
    yt = y_ref[s]
    nb = n_ref[s]

    d = None
    for c in range(3):
        diff = yt[c : c + 1, :] - xb[:, c : c + 1]
        sq = diff * diff
        d = sq if d is None else d + sq

    row_min = jnp.min(d, axis=1, keepdims=True)
    x2y_ref[s] = jnp.sqrt(row_min)

    col_min = jnp.min(d, axis=0, keepdims=True)
    onehot = jnp.where(d == col_min, 1.0, 0.0).astype(jnp.bfloat16)

    a = (
        nb[:, 0:1] * xb[:, 0:1]
        + nb[:, 1:2] * xb[:, 1:2]
        + nb[:, 2:3] * xb[:, 2:3]
    )
    ii = jax.lax.broadcasted_iota(jnp.int32, (_P1, 1), 0).astype(jnp.float32)
    ones = jnp.ones((_P1, 1), jnp.float32)
    table = jnp.concatenate([nb, a, ii, ones, ii * ii], axis=1)
    dims = (((0,), (0,)), ((), ()))
    gath = None
    for limb in _split3_bf16(table):
        part = jax.lax.dot_general(
            limb, onehot, dims, preferred_element_type=jnp.float32
        )
        gath = part if gath is None else gath + part

    ssum = gath[4:5]
    cnt = gath[5:6]
    q = gath[6:7]
    tie_lo = 0.5 * (ssum - jnp.sqrt(jnp.maximum(2.0 * q - ssum * ssum, 0.0)))
    yidx = jnp.where(cnt == 1.0, ssum, tie_lo)
    yidx_ref[s] = yidx.astype(jnp.int32)

    dotv = (
        gath[0:1] * yt[0:1]
        + gath[1:2] * yt[1:2]
        + gath[2:3] * yt[2:3]
        - gath[3:4]
    )
    y2x_ref[s] = jnp.sqrt(col_min) * jnp.sign(dotv)


@functools.partial(jax.jit, static_argnames=())
def kernel(x, y, x_normals):
    B = x.shape[0]
    yt = jnp.transpose(y, (0, 2, 1))

    y2x_s, x2y_s, yidx = pl.pallas_call(
        _nn_body,
        grid=(B // _BPP,),
        in_specs=[
            pl.BlockSpec((_BPP, _P1, 3), lambda b: (b, 0, 0)),
            pl.BlockSpec((_BPP, 3, _P2), lambda b: (b, 0, 0)),
            pl.BlockSpec((_BPP, _P1, 3), lambda b: (b, 0, 0)),
        ],
        out_specs=[
            pl.BlockSpec((_BPP, 1, _P2), lambda b: (b, 0, 0)),
            pl.BlockSpec((_BPP, _P1, 1), lambda b: (b, 0, 0)),
            pl.BlockSpec((_BPP, 1, _P2), lambda b: (b, 0, 0)),
        ],
        out_shape=[
            jax.ShapeDtypeStruct((B, 1, _P2), jnp.float32),
            jax.ShapeDtypeStruct((B, _P1, 1), jnp.float32),
            jax.ShapeDtypeStruct((B, 1, _P2), jnp.int32),
        ],
    )(x, yt, x_normals)

    return (
        y2x_s[:, 0, :],
        x2y_s[:, :, 0],
        yidx[:, 0, :],
    )
